# Initial kernel scaffold; baseline (speedup 1.0000x reference)
import jax
import jax.numpy as jnp
from jax.experimental import pallas as pl


def kernel(x, edge_attr, edge_length, edge_src, edge_dst, W1, W2, W_sc):
    raise NotImplementedError("write your pallas kernel here")


# speedup vs baseline: 3.4103x; 3.4103x over previous
import functools

import jax
import jax.numpy as jnp
import numpy as np
from jax import lax
from jax.experimental import pallas as pl
from jax.experimental.pallas import tpu as pltpu
from jax.experimental.pallas import tpu_sc as plsc

N = 10000
E = 320000
MUL = 16
NUM_RADIAL = 8
HIDDEN = 64
WEIGHT_NUMEL = MUL * MUL

NC = 2
NS = 16
NW = NC * NS

CH = 128
NCHUNK = E // CH
GATHER_TRIPS = -(-NCHUNK // NW)
E_HALF = E // 2
NCH_CORE = E_HALF // CH
SCAT_TRIPS = -(-NCH_CORE // NS)
ROWS_PER_TILE = N // NS

_mesh = plsc.VectorSubcoreMesh(core_axis_name="c", subcore_axis_name="s")
_sc_params = pltpu.CompilerParams(use_tc_tiling_on_sc=False)


@functools.partial(
    pl.kernel,
    mesh=_mesh,
    out_type=jax.ShapeDtypeStruct((E, MUL), jnp.float32),
    scratch_types=[
        pltpu.VMEM((CH,), jnp.int32),
        pltpu.VMEM((CH, MUL), jnp.float32),
        pltpu.SemaphoreType.DMA,
    ],
    compiler_params=_sc_params,
)
def _gather_k(x_hbm, src_hbm, xj_hbm, idx_v, rows_v, sem):
    wid = lax.axis_index("s") * NC + lax.axis_index("c")

    def body(i, carry):
        j = wid + i * NW

        @pl.when(j < NCHUNK)
        def _():
            base = j * CH
            pltpu.sync_copy(src_hbm.at[pl.ds(base, CH)], idx_v)
            pltpu.async_copy(x_hbm.at[idx_v], rows_v, sem).wait()
            pltpu.sync_copy(rows_v, xj_hbm.at[pl.ds(base, CH)])

        return carry

    lax.fori_loop(0, GATHER_TRIPS, body, 0)


@functools.partial(
    pl.kernel,
    mesh=_mesh,
    out_type=jax.ShapeDtypeStruct((2 * N, MUL), jnp.float32),
    scratch_types=[
        pltpu.VMEM((CH,), jnp.int32),
        pltpu.VMEM((CH, MUL), jnp.float32),
        pltpu.VMEM_SHARED((N, MUL), jnp.float32),
        pltpu.SemaphoreType.DMA,
    ],
    compiler_params=_sc_params,
)
def _scatter_k(m_hbm, dst_hbm, zero_hbm, out_hbm, idx_v, rows_v, acc_sh, sem):
    cid = lax.axis_index("c")
    sid = lax.axis_index("s")
    r0 = sid * ROWS_PER_TILE
    pltpu.sync_copy(zero_hbm.at[pl.ds(r0, ROWS_PER_TILE)],
                    acc_sh.at[pl.ds(r0, ROWS_PER_TILE)])
    plsc.subcore_barrier()

    def body(i, carry):
        j = sid + i * NS

        @pl.when(j < NCH_CORE)
        def _():
            base = cid * E_HALF + j * CH
            pltpu.sync_copy(dst_hbm.at[pl.ds(base, CH)], idx_v)
            pltpu.sync_copy(m_hbm.at[pl.ds(base, CH)], rows_v)
            pltpu.sync_copy(rows_v, acc_sh.at[idx_v], add=True)

        return carry

    lax.fori_loop(0, SCAT_TRIPS, body, 0)
    plsc.subcore_barrier()
    pltpu.sync_copy(acc_sh.at[pl.ds(r0, ROWS_PER_TILE)],
                    out_hbm.at[pl.ds(cid * N + r0, ROWS_PER_TILE)])


_BLK = 2560


def _main_body(el_ref, ea_ref, xj_ref, w1_ref, w2_ref, o_ref):
    el = el_ref[...]
    centers = lax.broadcasted_iota(
        jnp.int32, (1, NUM_RADIAL), 1).astype(jnp.float32) * np.float32(5.0 / 7.0)
    d = el - centers
    radial = jnp.exp(-0.5 * d * d)
    w1 = w1_ref[...] * np.float32(1.0 / np.sqrt(NUM_RADIAL))
    h = jnp.dot(radial, w1, preferred_element_type=jnp.float32)
    h = h / (1.0 + jnp.exp(-h))
    w2 = w2_ref[...] * np.float32(1.0 / np.sqrt(HIDDEN))
    wts = jnp.dot(h, w2, preferred_element_type=jnp.float32)

    xj = xj_ref[...]
    u_t = lax.broadcasted_iota(jnp.int32, (MUL, WEIGHT_NUMEL), 0)
    c_t = lax.broadcasted_iota(jnp.int32, (MUL, WEIGHT_NUMEL), 1)
    tile_m = jnp.where(c_t % MUL == u_t, 1.0, 0.0).astype(jnp.float32)
    xt = jnp.dot(xj, tile_m, preferred_element_type=jnp.float32)
    p = wts * xt
    r_s = lax.broadcasted_iota(jnp.int32, (WEIGHT_NUMEL, MUL), 0)
    w_s = lax.broadcasted_iota(jnp.int32, (WEIGHT_NUMEL, MUL), 1)
    seg_m = jnp.where(r_s // MUL == w_s, 1.0, 0.0).astype(jnp.float32)
    m = jnp.dot(p, seg_m, preferred_element_type=jnp.float32)
    o_ref[...] = m * ea_ref[...] * np.float32(1.0 / np.sqrt(MUL))


def _tc_main(el2, ea, xj, W1, W2):
    grid = (E // _BLK,)
    return pl.pallas_call(
        _main_body,
        grid=grid,
        in_specs=[
            pl.BlockSpec((_BLK, 1), lambda i: (i, 0)),
            pl.BlockSpec((_BLK, 1), lambda i: (i, 0)),
            pl.BlockSpec((_BLK, MUL), lambda i: (i, 0)),
            pl.BlockSpec((NUM_RADIAL, HIDDEN), lambda i: (0, 0)),
            pl.BlockSpec((HIDDEN, WEIGHT_NUMEL), lambda i: (0, 0)),
        ],
        out_specs=pl.BlockSpec((_BLK, MUL), lambda i: (i, 0)),
        out_shape=jax.ShapeDtypeStruct((E, MUL), jnp.float32),
    )(el2, ea, xj, W1, W2)


def _comb_body(p_ref, x_ref, wsc_ref, o_ref):
    wsc = wsc_ref[...] * np.float32(1.0 / np.sqrt(MUL))
    sc = jnp.dot(x_ref[...], wsc, preferred_element_type=jnp.float32)
    o_ref[...] = p_ref[0] + p_ref[1] + sc


def _tc_combine(partials, x, W_sc):
    return pl.pallas_call(
        _comb_body,
        out_shape=jax.ShapeDtypeStruct((N, MUL), jnp.float32),
    )(partials, x, W_sc)


def kernel(x, edge_attr, edge_length, edge_src, edge_dst, W1, W2, W_sc):
    src = edge_src.astype(jnp.int32)
    dst = edge_dst.astype(jnp.int32)
    x_j = _gather_k(x, src)
    el2 = edge_length.reshape(E, 1)
    m_ij = _tc_main(el2, edge_attr, x_j, W1, W2)
    zeros = jnp.zeros((N, MUL), dtype=jnp.float32)
    partials = _scatter_k(m_ij, dst, zeros)
    partials = partials.reshape(2, N, MUL)
    return _tc_combine(partials, x, W_sc)
